# baseline (speedup 1.0000x reference)
import functools

import jax
import jax.numpy as jnp
from jax import lax
from jax.experimental import pallas as pl
from jax.experimental.pallas import tpu as pltpu
from jax.experimental.pallas import tpu_sc as plsc

B = 16384
L = 200
E = 32
V = 1000000
NCLS = 50
NCLS_PAD = 64
NW = 32
BPW = B // NW

_mesh = plsc.VectorSubcoreMesh(core_axis_name="c", subcore_axis_name="s")

DT_CR = 192
DT_VB = 4 * DT_CR
DT_FLAT = DT_VB * E
DT_NFULL = (V // 4) // DT_CR
DT_TAIL_V = V - DT_NFULL * DT_VB
_XBP = 775


@functools.partial(
    pl.kernel,
    out_type=jax.ShapeDtypeStruct((V * E,), jnp.float32),
    mesh=_mesh,
    scratch_types=[
        pltpu.VMEM((32, _XBP), jnp.float32),
        pltpu.VMEM((32, _XBP), jnp.float32),
        pltpu.VMEM((DT_FLAT,), jnp.float32),
        pltpu.VMEM((DT_FLAT,), jnp.float32),
        pltpu.SemaphoreType.DMA,
        pltpu.SemaphoreType.DMA,
        pltpu.SemaphoreType.DMA,
        pltpu.SemaphoreType.DMA,
    ],
    compiler_params=pltpu.CompilerParams(
        use_tc_tiling_on_sc=True, needs_layout_passes=False),
)
def _detile_sc(tabT_hbm, tail_hbm, out_hbm, xb0, xb1, ob0, ob1, si0, si1, so0, so1):
    w = lax.axis_index("s") * 2 + lax.axis_index("c")
    iota = lax.iota(jnp.int32, 16)

    def fire_in(c, xb, sem):
        pltpu.make_async_copy(
            tabT_hbm.at[:, pl.ds(c * DT_VB, DT_VB)],
            xb.at[:, pl.ds(0, DT_VB)], sem).start()

    def wait_in(xb, sem):
        pltpu.make_async_copy(
            tabT_hbm.at[:, pl.ds(0, DT_VB)],
            xb.at[:, pl.ds(0, DT_VB)], sem).wait()

    def fire_out(c, ob, sem):
        pltpu.make_async_copy(ob, out_hbm.at[pl.ds(c * DT_FLAT, DT_FLAT)], sem).start()

    def wait_out(ob, sem):
        pltpu.make_async_copy(ob, out_hbm.at[pl.ds(0, DT_FLAT)], sem).wait()

    def transpose(xb, ob, n_i):
        def tbody(i, carry):
            for e0 in (0, 16):
                vals = plsc.load_gather(xb, [e0 + iota, jnp.full((16,), 0, jnp.int32) + i])
                ob[pl.ds(i * 32 + e0, 16)] = vals
            return carry
        lax.fori_loop(0, n_i, tbody, 0)

    fire_in(w, xb0, si0)
    fire_in(w + 32, xb1, si1)

    def body(k2, carry):
        for p, xb, ob, si, so in ((0, xb0, ob0, si0, so0), (1, xb1, ob1, si1, so1)):
            k = 2 * k2 + p
            c = w + 32 * k

            @pl.when(c < DT_NFULL)
            def _():
                wait_in(xb, si)

                @pl.when(k2 > 0)
                def _():
                    wait_out(ob, so)

                transpose(xb, ob, DT_VB)
                fire_out(c, ob, so)

                @pl.when(w + 32 * (k + 2) < DT_NFULL)
                def _():
                    fire_in(w + 32 * (k + 2), xb, si)
        return carry

    lax.fori_loop(0, (DT_NFULL + 63) // 64 + 1, body, 0)
    wait_out(ob0, so0)
    wait_out(ob1, so1)

    @pl.when(w == 31)
    def _():
        cp = pltpu.make_async_copy(tail_hbm, ob0.at[pl.ds(0, DT_TAIL_V * E)], si0)
        cp.start()
        cp.wait()
        cpo = pltpu.make_async_copy(
            ob0.at[pl.ds(0, DT_TAIL_V * E)],
            out_hbm.at[pl.ds(DT_NFULL * DT_FLAT, DT_TAIL_V * E)], so0)
        cpo.start()
        cpo.wait()


R = 8
NCHUNK = BPW // R
_SPLITS = ((0, 128), (128, 72))


@functools.partial(
    pl.kernel,
    out_type=jax.ShapeDtypeStruct((B, E), jnp.float32),
    mesh=_mesh,
    scratch_types=[
        pltpu.VMEM((R, L), jnp.int32),
        pltpu.VMEM((R, L), jnp.int32),
        pltpu.VMEM((R, L, E), jnp.float32),
        pltpu.VMEM((R, L, E), jnp.float32),
        pltpu.VMEM((BPW, E), jnp.float32),
        pltpu.SemaphoreType.DMA,
        pltpu.SemaphoreType.DMA,
        pltpu.SemaphoreType.DMA,
    ],
    compiler_params=pltpu.CompilerParams(use_tc_tiling_on_sc=False),
)
def _pool_sc(ids_hbm, table_hbm, out_hbm, ibuf0, ibuf1, rbuf0, rbuf1,
             obuf, sem_i, sem_g0, sem_g1):
    wid = lax.axis_index("s") * 2 + lax.axis_index("c")
    base = wid * BPW

    def fire_idx(c, ibuf):
        pltpu.make_async_copy(ids_hbm.at[pl.ds(base + c * R, R)], ibuf, sem_i).start()

    def wait_idx(ibuf):
        pltpu.make_async_copy(ids_hbm.at[pl.ds(base, R)], ibuf, sem_i).wait()

    def fire_gathers(ibuf, rbuf, sem):
        for r in range(R):
            for (o, w) in _SPLITS:
                pltpu.make_async_copy(
                    table_hbm.at[ibuf.at[r, pl.ds(o, w)]],
                    rbuf.at[r, pl.ds(o, w)], sem).start()

    def wait_gathers(ibuf, rbuf, sem):
        for r in range(R):
            for (o, w) in _SPLITS:
                pltpu.make_async_copy(
                    table_hbm.at[ibuf.at[r, pl.ds(o, w)]],
                    rbuf.at[r, pl.ds(o, w)], sem).wait()

    def reduce_chunk(c, rbuf):
        for r in range(R):
            def red(j, accs):
                a0, a1 = accs
                return a0 + rbuf[r, j, pl.ds(0, 16)], a1 + rbuf[r, j, pl.ds(16, 16)]
            z = jnp.zeros((16,), jnp.float32)
            a0, a1 = lax.fori_loop(0, L, red, (z, z))
            row = c * R + r
            obuf[row, pl.ds(0, 16)] = a0
            obuf[row, pl.ds(16, 16)] = a1

    fire_idx(0, ibuf0)

    def body(c2, carry):
        c = 2 * c2
        wait_idx(ibuf0)
        fire_gathers(ibuf0, rbuf0, sem_g0)

        @pl.when(c2 > 0)
        def _():
            wait_gathers(ibuf1, rbuf1, sem_g1)

        fire_idx(c + 1, ibuf1)

        @pl.when(c2 > 0)
        def _():
            reduce_chunk(c - 1, rbuf1)

        wait_idx(ibuf1)
        fire_gathers(ibuf1, rbuf1, sem_g1)
        wait_gathers(ibuf0, rbuf0, sem_g0)

        @pl.when(c2 < NCHUNK // 2 - 1)
        def _():
            fire_idx(c + 2, ibuf0)

        reduce_chunk(c, rbuf0)
        return carry

    lax.fori_loop(0, NCHUNK // 2, body, 0)
    wait_gathers(ibuf1, rbuf1, sem_g1)
    reduce_chunk(NCHUNK - 1, rbuf1)
    pltpu.sync_copy(obuf, out_hbm.at[pl.ds(base, BPW)])


def _mlp_body(x_ref, w1_ref, b1_ref, w2_ref, b2_ref, o_ref):
    h = jnp.dot(x_ref[...], w1_ref[...], preferred_element_type=jnp.float32)
    h = jnp.maximum(h + b1_ref[...], 0.0)
    o_ref[...] = jnp.dot(h, w2_ref[...], preferred_element_type=jnp.float32) + b2_ref[...]


_BM = 2048

_mlp = pl.pallas_call(
    _mlp_body,
    grid=(B // _BM,),
    in_specs=[
        pl.BlockSpec((_BM, E), lambda i: (i, 0)),
        pl.BlockSpec((E, 128), lambda i: (0, 0)),
        pl.BlockSpec((1, 128), lambda i: (0, 0)),
        pl.BlockSpec((128, NCLS_PAD), lambda i: (0, 0)),
        pl.BlockSpec((1, NCLS_PAD), lambda i: (0, 0)),
    ],
    out_specs=pl.BlockSpec((_BM, NCLS_PAD), lambda i: (i, 0)),
    out_shape=jax.ShapeDtypeStruct((B, NCLS_PAD), jnp.float32),
)


def kernel(input_ids, table, W1, b1, W2, b2):
    tail = table[DT_NFULL * DT_VB:].reshape(-1)
    lin = _detile_sc(table.T, tail)
    pooled = _pool_sc(input_ids.astype(jnp.int32), lin.reshape(V, E))
    w1s = W1.T.astype(jnp.float32) * (1.0 / L)
    b1r = b1.reshape(1, 128)
    w2p = jnp.pad(W2.T, ((0, 0), (0, NCLS_PAD - NCLS)))
    b2p = jnp.pad(b2, (0, NCLS_PAD - NCLS)).reshape(1, NCLS_PAD)
    out = _mlp(pooled, w1s, b1r, w2p, b2p)
    return out[:, :NCLS]


# speedup vs baseline: 1.3366x; 1.3366x over previous
import functools

import jax
import jax.numpy as jnp
from jax import lax
from jax.experimental import pallas as pl
from jax.experimental.pallas import tpu as pltpu
from jax.experimental.pallas import tpu_sc as plsc

B = 16384
L = 200
E = 32
V = 1000000
NCLS = 50
NCLS_PAD = 64
NW = 32
BPW = B // NW

_mesh = plsc.VectorSubcoreMesh(core_axis_name="c", subcore_axis_name="s")

DT_CR = 192
DT_VB = 4 * DT_CR
DT_FLAT = DT_VB * E
DT_NFULL = (V // 4) // DT_CR
DT_TAIL_V = V - DT_NFULL * DT_VB
_XBP = 775


@functools.partial(
    pl.kernel,
    out_type=jax.ShapeDtypeStruct((V * E,), jnp.float32),
    mesh=_mesh,
    scratch_types=[
        pltpu.VMEM((32, _XBP), jnp.float32),
        pltpu.VMEM((32, _XBP), jnp.float32),
        pltpu.VMEM((DT_FLAT,), jnp.float32),
        pltpu.VMEM((DT_FLAT,), jnp.float32),
        pltpu.SemaphoreType.DMA,
        pltpu.SemaphoreType.DMA,
        pltpu.SemaphoreType.DMA,
        pltpu.SemaphoreType.DMA,
    ],
    compiler_params=pltpu.CompilerParams(
        use_tc_tiling_on_sc=True, needs_layout_passes=False),
)
def _detile_sc(tabT_hbm, tail_hbm, out_hbm, xb0, xb1, ob0, ob1, si0, si1, so0, so1):
    w = lax.axis_index("s") * 2 + lax.axis_index("c")
    iota = lax.iota(jnp.int32, 16)

    def fire_in(c, xb, sem):
        pltpu.make_async_copy(
            tabT_hbm.at[:, pl.ds(c * DT_VB, DT_VB)],
            xb.at[:, pl.ds(0, DT_VB)], sem).start()

    def wait_in(xb, sem):
        pltpu.make_async_copy(
            tabT_hbm.at[:, pl.ds(0, DT_VB)],
            xb.at[:, pl.ds(0, DT_VB)], sem).wait()

    def fire_out(c, ob, sem):
        pltpu.make_async_copy(ob, out_hbm.at[pl.ds(c * DT_FLAT, DT_FLAT)], sem).start()

    def wait_out(ob, sem):
        pltpu.make_async_copy(ob, out_hbm.at[pl.ds(0, DT_FLAT)], sem).wait()

    e_lo = iota
    e_hi = iota + 16
    zero16 = iota * 0

    def transpose(xb, ob, n_i):
        def tbody(ii, carry):
            base = ii * 8
            for u in range(8):
                i = base + u
                col = zero16 + i
                v0 = plsc.load_gather(xb, [e_lo, col])
                v1 = plsc.load_gather(xb, [e_hi, col])
                ob[pl.ds(i * 32, 16)] = v0
                ob[pl.ds(i * 32 + 16, 16)] = v1
            return carry
        lax.fori_loop(0, n_i // 8, tbody, 0)

    fire_in(w, xb0, si0)
    fire_in(w + 32, xb1, si1)

    def body(k2, carry):
        for p, xb, ob, si, so in ((0, xb0, ob0, si0, so0), (1, xb1, ob1, si1, so1)):
            k = 2 * k2 + p
            c = w + 32 * k

            @pl.when(c < DT_NFULL)
            def _():
                wait_in(xb, si)

                @pl.when(k2 > 0)
                def _():
                    wait_out(ob, so)

                transpose(xb, ob, DT_VB)
                fire_out(c, ob, so)

                @pl.when(w + 32 * (k + 2) < DT_NFULL)
                def _():
                    fire_in(w + 32 * (k + 2), xb, si)
        return carry

    lax.fori_loop(0, (DT_NFULL + 63) // 64 + 1, body, 0)
    wait_out(ob0, so0)
    wait_out(ob1, so1)

    @pl.when(w == 31)
    def _():
        cp = pltpu.make_async_copy(tail_hbm, ob0.at[pl.ds(0, DT_TAIL_V * E)], si0)
        cp.start()
        cp.wait()
        cpo = pltpu.make_async_copy(
            ob0.at[pl.ds(0, DT_TAIL_V * E)],
            out_hbm.at[pl.ds(DT_NFULL * DT_FLAT, DT_TAIL_V * E)], so0)
        cpo.start()
        cpo.wait()


R = 8
NCHUNK = BPW // R
_SPLITS = ((0, 128), (128, 72))


@functools.partial(
    pl.kernel,
    out_type=jax.ShapeDtypeStruct((B, E), jnp.float32),
    mesh=_mesh,
    scratch_types=[
        pltpu.VMEM((R, L), jnp.int32),
        pltpu.VMEM((R, L), jnp.int32),
        pltpu.VMEM((R, L, E), jnp.float32),
        pltpu.VMEM((R, L, E), jnp.float32),
        pltpu.VMEM((BPW, E), jnp.float32),
        pltpu.SemaphoreType.DMA,
        pltpu.SemaphoreType.DMA,
        pltpu.SemaphoreType.DMA,
    ],
    compiler_params=pltpu.CompilerParams(use_tc_tiling_on_sc=False),
)
def _pool_sc(ids_hbm, table_hbm, out_hbm, ibuf0, ibuf1, rbuf0, rbuf1,
             obuf, sem_i, sem_g0, sem_g1):
    wid = lax.axis_index("s") * 2 + lax.axis_index("c")
    base = wid * BPW

    def fire_idx(c, ibuf):
        pltpu.make_async_copy(ids_hbm.at[pl.ds(base + c * R, R)], ibuf, sem_i).start()

    def wait_idx(ibuf):
        pltpu.make_async_copy(ids_hbm.at[pl.ds(base, R)], ibuf, sem_i).wait()

    def fire_gathers(ibuf, rbuf, sem):
        for r in range(R):
            for (o, w) in _SPLITS:
                pltpu.make_async_copy(
                    table_hbm.at[ibuf.at[r, pl.ds(o, w)]],
                    rbuf.at[r, pl.ds(o, w)], sem).start()

    def wait_gathers(ibuf, rbuf, sem):
        for r in range(R):
            for (o, w) in _SPLITS:
                pltpu.make_async_copy(
                    table_hbm.at[ibuf.at[r, pl.ds(o, w)]],
                    rbuf.at[r, pl.ds(o, w)], sem).wait()

    def reduce_chunk(c, rbuf):
        def red(j, accs):
            out = []
            for r in range(R):
                out.append(accs[2 * r] + rbuf[r, j, pl.ds(0, 16)])
                out.append(accs[2 * r + 1] + rbuf[r, j, pl.ds(16, 16)])
            return tuple(out)
        z = jnp.zeros((16,), jnp.float32)
        accs = lax.fori_loop(0, L, red, (z,) * (2 * R))
        for r in range(R):
            row = c * R + r
            obuf[row, pl.ds(0, 16)] = accs[2 * r]
            obuf[row, pl.ds(16, 16)] = accs[2 * r + 1]

    fire_idx(0, ibuf0)

    def body(c2, carry):
        c = 2 * c2
        wait_idx(ibuf0)
        fire_gathers(ibuf0, rbuf0, sem_g0)

        @pl.when(c2 > 0)
        def _():
            wait_gathers(ibuf1, rbuf1, sem_g1)

        fire_idx(c + 1, ibuf1)

        @pl.when(c2 > 0)
        def _():
            reduce_chunk(c - 1, rbuf1)

        wait_idx(ibuf1)
        fire_gathers(ibuf1, rbuf1, sem_g1)
        wait_gathers(ibuf0, rbuf0, sem_g0)

        @pl.when(c2 < NCHUNK // 2 - 1)
        def _():
            fire_idx(c + 2, ibuf0)

        reduce_chunk(c, rbuf0)
        return carry

    lax.fori_loop(0, NCHUNK // 2, body, 0)
    wait_gathers(ibuf1, rbuf1, sem_g1)
    reduce_chunk(NCHUNK - 1, rbuf1)
    pltpu.sync_copy(obuf, out_hbm.at[pl.ds(base, BPW)])


def _mlp_body(x_ref, w1_ref, b1_ref, w2_ref, b2_ref, o_ref):
    h = jnp.dot(x_ref[...], w1_ref[...], preferred_element_type=jnp.float32)
    h = jnp.maximum(h + b1_ref[...], 0.0)
    o_ref[...] = jnp.dot(h, w2_ref[...], preferred_element_type=jnp.float32) + b2_ref[...]


_BM = 2048

_mlp = pl.pallas_call(
    _mlp_body,
    grid=(B // _BM,),
    in_specs=[
        pl.BlockSpec((_BM, E), lambda i: (i, 0)),
        pl.BlockSpec((E, 128), lambda i: (0, 0)),
        pl.BlockSpec((1, 128), lambda i: (0, 0)),
        pl.BlockSpec((128, NCLS_PAD), lambda i: (0, 0)),
        pl.BlockSpec((1, NCLS_PAD), lambda i: (0, 0)),
    ],
    out_specs=pl.BlockSpec((_BM, NCLS_PAD), lambda i: (i, 0)),
    out_shape=jax.ShapeDtypeStruct((B, NCLS_PAD), jnp.float32),
)


def kernel(input_ids, table, W1, b1, W2, b2):
    tail = table[DT_NFULL * DT_VB:].reshape(-1)
    lin = _detile_sc(table.T, tail)
    pooled = _pool_sc(input_ids.astype(jnp.int32), lin.reshape(V, E))
    w1s = W1.T.astype(jnp.float32) * (1.0 / L)
    b1r = b1.reshape(1, 128)
    w2p = jnp.pad(W2.T, ((0, 0), (0, NCLS_PAD - NCLS)))
    b2p = jnp.pad(b2, (0, NCLS_PAD - NCLS)).reshape(1, NCLS_PAD)
    out = _mlp(pooled, w1s, b1r, w2p, b2p)
    return out[:, :NCLS]


# speedup vs baseline: 3.2567x; 2.4365x over previous
import functools

import jax
import jax.numpy as jnp
from jax import lax
from jax.experimental import pallas as pl
from jax.experimental.pallas import tpu as pltpu
from jax.experimental.pallas import tpu_sc as plsc

B = 16384
L = 200
E = 32
V = 1000000
NCLS = 50
NCLS_PAD = 64
NW = 32
BPW = B // NW

_mesh = plsc.VectorSubcoreMesh(core_axis_name="c", subcore_axis_name="s")

DT_CR = 192
DT_VB = 4 * DT_CR
DT_FLAT = DT_VB * E
DT_NFULL = (V // 4) // DT_CR
DT_TAIL_V = V - DT_NFULL * DT_VB
_XBP = 775


@functools.partial(
    pl.kernel,
    out_type=jax.ShapeDtypeStruct((V * E,), jnp.float32),
    mesh=_mesh,
    scratch_types=[
        pltpu.VMEM((32, _XBP), jnp.float32),
        pltpu.VMEM((32, _XBP), jnp.float32),
        pltpu.VMEM((DT_FLAT,), jnp.float32),
        pltpu.VMEM((DT_FLAT,), jnp.float32),
        pltpu.SemaphoreType.DMA,
        pltpu.SemaphoreType.DMA,
        pltpu.SemaphoreType.DMA,
        pltpu.SemaphoreType.DMA,
    ],
    compiler_params=pltpu.CompilerParams(
        use_tc_tiling_on_sc=True, needs_layout_passes=False),
)
def _detile_sc(tabT_hbm, tail_hbm, out_hbm, xb0, xb1, ob0, ob1, si0, si1, so0, so1):
    w = lax.axis_index("s") * 2 + lax.axis_index("c")
    iota = lax.iota(jnp.int32, 16)

    def fire_in(c, xb, sem):
        pltpu.make_async_copy(
            tabT_hbm.at[:, pl.ds(c * DT_VB, DT_VB)],
            xb.at[:, pl.ds(0, DT_VB)], sem).start()

    def wait_in(xb, sem):
        pltpu.make_async_copy(
            tabT_hbm.at[:, pl.ds(0, DT_VB)],
            xb.at[:, pl.ds(0, DT_VB)], sem).wait()

    def fire_out(c, ob, sem):
        pltpu.make_async_copy(ob, out_hbm.at[pl.ds(c * DT_FLAT, DT_FLAT)], sem).start()

    def wait_out(ob, sem):
        pltpu.make_async_copy(ob, out_hbm.at[pl.ds(0, DT_FLAT)], sem).wait()

    perm_idx = {s: iota ^ s for s in (1, 2, 4, 8)}
    mask_eq = {s: (iota & s) == 0 for s in (1, 2, 4, 8)}
    mask_ne = {s: (iota & s) != 0 for s in (1, 2, 4, 8)}

    def transpose16(xb, ob, i0, e0):
        r = [xb[e0 + e, pl.ds(i0, 16)] for e in range(16)]
        for s in (1, 2, 4, 8):
            idx = perm_idx[s]
            nxt = []
            for v in range(16):
                tmp = r[v ^ s].at[idx].get(mode="promise_in_bounds")
                keep = mask_eq[s] if (v & s) == 0 else mask_ne[s]
                nxt.append(jnp.where(keep, r[v], tmp))
            r = nxt
        for v in range(16):
            ob[pl.ds((i0 + v) * 32 + e0, 16)] = r[v]

    def transpose(xb, ob, n_i):
        def tbody(ii, carry):
            i0 = ii * 16
            transpose16(xb, ob, i0, 0)
            transpose16(xb, ob, i0, 16)
            return carry
        lax.fori_loop(0, n_i // 16, tbody, 0)

    fire_in(w, xb0, si0)
    fire_in(w + 32, xb1, si1)

    def body(k2, carry):
        for p, xb, ob, si, so in ((0, xb0, ob0, si0, so0), (1, xb1, ob1, si1, so1)):
            k = 2 * k2 + p
            c = w + 32 * k

            @pl.when(c < DT_NFULL)
            def _():
                wait_in(xb, si)

                @pl.when(k2 > 0)
                def _():
                    wait_out(ob, so)

                transpose(xb, ob, DT_VB)
                fire_out(c, ob, so)

                @pl.when(w + 32 * (k + 2) < DT_NFULL)
                def _():
                    fire_in(w + 32 * (k + 2), xb, si)
        return carry

    lax.fori_loop(0, (DT_NFULL + 63) // 64 + 1, body, 0)
    wait_out(ob0, so0)
    wait_out(ob1, so1)

    @pl.when(w == 31)
    def _():
        cp = pltpu.make_async_copy(tail_hbm, ob0.at[pl.ds(0, DT_TAIL_V * E)], si0)
        cp.start()
        cp.wait()
        cpo = pltpu.make_async_copy(
            ob0.at[pl.ds(0, DT_TAIL_V * E)],
            out_hbm.at[pl.ds(DT_NFULL * DT_FLAT, DT_TAIL_V * E)], so0)
        cpo.start()
        cpo.wait()


R = 8
NCHUNK = BPW // R
_SPLITS = ((0, 128), (128, 72))


@functools.partial(
    pl.kernel,
    out_type=jax.ShapeDtypeStruct((B, E), jnp.float32),
    mesh=_mesh,
    scratch_types=[
        pltpu.VMEM((R, L), jnp.int32),
        pltpu.VMEM((R, L), jnp.int32),
        pltpu.VMEM((R, L, E), jnp.float32),
        pltpu.VMEM((R, L, E), jnp.float32),
        pltpu.VMEM((BPW, E), jnp.float32),
        pltpu.SemaphoreType.DMA,
        pltpu.SemaphoreType.DMA,
        pltpu.SemaphoreType.DMA,
    ],
    compiler_params=pltpu.CompilerParams(use_tc_tiling_on_sc=False),
)
def _pool_sc(ids_hbm, table_hbm, out_hbm, ibuf0, ibuf1, rbuf0, rbuf1,
             obuf, sem_i, sem_g0, sem_g1):
    wid = lax.axis_index("s") * 2 + lax.axis_index("c")
    base = wid * BPW

    def fire_idx(c, ibuf):
        pltpu.make_async_copy(ids_hbm.at[pl.ds(base + c * R, R)], ibuf, sem_i).start()

    def wait_idx(ibuf):
        pltpu.make_async_copy(ids_hbm.at[pl.ds(base, R)], ibuf, sem_i).wait()

    def fire_gathers(ibuf, rbuf, sem):
        for r in range(R):
            for (o, w) in _SPLITS:
                pltpu.make_async_copy(
                    table_hbm.at[ibuf.at[r, pl.ds(o, w)]],
                    rbuf.at[r, pl.ds(o, w)], sem).start()

    def wait_gathers(ibuf, rbuf, sem):
        for r in range(R):
            for (o, w) in _SPLITS:
                pltpu.make_async_copy(
                    table_hbm.at[ibuf.at[r, pl.ds(o, w)]],
                    rbuf.at[r, pl.ds(o, w)], sem).wait()

    def reduce_chunk(c, rbuf):
        def red(j, accs):
            out = []
            for r in range(R):
                out.append(accs[2 * r] + rbuf[r, j, pl.ds(0, 16)])
                out.append(accs[2 * r + 1] + rbuf[r, j, pl.ds(16, 16)])
            return tuple(out)
        z = jnp.zeros((16,), jnp.float32)
        accs = lax.fori_loop(0, L, red, (z,) * (2 * R))
        for r in range(R):
            row = c * R + r
            obuf[row, pl.ds(0, 16)] = accs[2 * r]
            obuf[row, pl.ds(16, 16)] = accs[2 * r + 1]

    fire_idx(0, ibuf0)

    def body(c2, carry):
        c = 2 * c2
        wait_idx(ibuf0)
        fire_gathers(ibuf0, rbuf0, sem_g0)

        @pl.when(c2 > 0)
        def _():
            wait_gathers(ibuf1, rbuf1, sem_g1)

        fire_idx(c + 1, ibuf1)

        @pl.when(c2 > 0)
        def _():
            reduce_chunk(c - 1, rbuf1)

        wait_idx(ibuf1)
        fire_gathers(ibuf1, rbuf1, sem_g1)
        wait_gathers(ibuf0, rbuf0, sem_g0)

        @pl.when(c2 < NCHUNK // 2 - 1)
        def _():
            fire_idx(c + 2, ibuf0)

        reduce_chunk(c, rbuf0)
        return carry

    lax.fori_loop(0, NCHUNK // 2, body, 0)
    wait_gathers(ibuf1, rbuf1, sem_g1)
    reduce_chunk(NCHUNK - 1, rbuf1)
    pltpu.sync_copy(obuf, out_hbm.at[pl.ds(base, BPW)])


def _mlp_body(x_ref, w1_ref, b1_ref, w2_ref, b2_ref, o_ref):
    h = jnp.dot(x_ref[...], w1_ref[...], preferred_element_type=jnp.float32)
    h = jnp.maximum(h + b1_ref[...], 0.0)
    o_ref[...] = jnp.dot(h, w2_ref[...], preferred_element_type=jnp.float32) + b2_ref[...]


_BM = 2048

_mlp = pl.pallas_call(
    _mlp_body,
    grid=(B // _BM,),
    in_specs=[
        pl.BlockSpec((_BM, E), lambda i: (i, 0)),
        pl.BlockSpec((E, 128), lambda i: (0, 0)),
        pl.BlockSpec((1, 128), lambda i: (0, 0)),
        pl.BlockSpec((128, NCLS_PAD), lambda i: (0, 0)),
        pl.BlockSpec((1, NCLS_PAD), lambda i: (0, 0)),
    ],
    out_specs=pl.BlockSpec((_BM, NCLS_PAD), lambda i: (i, 0)),
    out_shape=jax.ShapeDtypeStruct((B, NCLS_PAD), jnp.float32),
)


def kernel(input_ids, table, W1, b1, W2, b2):
    tail = table[DT_NFULL * DT_VB:].reshape(-1)
    lin = _detile_sc(table.T, tail)
    pooled = _pool_sc(input_ids.astype(jnp.int32), lin.reshape(V, E))
    w1s = W1.T.astype(jnp.float32) * (1.0 / L)
    b1r = b1.reshape(1, 128)
    w2p = jnp.pad(W2.T, ((0, 0), (0, NCLS_PAD - NCLS)))
    b2p = jnp.pad(b2, (0, NCLS_PAD - NCLS)).reshape(1, NCLS_PAD)
    out = _mlp(pooled, w1s, b1r, w2p, b2p)
    return out[:, :NCLS]


# speedup vs baseline: 3.5544x; 1.0914x over previous
import functools

import jax
import jax.numpy as jnp
from jax import lax
from jax.experimental import pallas as pl
from jax.experimental.pallas import tpu as pltpu
from jax.experimental.pallas import tpu_sc as plsc

B = 16384
L = 200
E = 32
V = 1000000
NCLS = 50
NCLS_PAD = 64
NW = 32
BPW = B // NW

_mesh = plsc.VectorSubcoreMesh(core_axis_name="c", subcore_axis_name="s")

DT_CR = 192
DT_VB = 4 * DT_CR
DT_FLAT = DT_VB * E
DT_NFULL = (V // 4) // DT_CR
DT_TAIL_V = V - DT_NFULL * DT_VB
_XBP = 775


@functools.partial(
    pl.kernel,
    out_type=jax.ShapeDtypeStruct((V * E // 2,), jnp.float32),
    mesh=_mesh,
    scratch_types=[
        pltpu.VMEM((32, _XBP), jnp.float32),
        pltpu.VMEM((32, _XBP), jnp.float32),
        pltpu.VMEM((DT_FLAT // 2,), jnp.float32),
        pltpu.VMEM((DT_FLAT // 2,), jnp.float32),
        pltpu.SemaphoreType.DMA,
        pltpu.SemaphoreType.DMA,
        pltpu.SemaphoreType.DMA,
        pltpu.SemaphoreType.DMA,
    ],
    compiler_params=pltpu.CompilerParams(
        use_tc_tiling_on_sc=True, needs_layout_passes=False),
)
def _detile_sc(tabT_hbm, tail_hbm, out_hbm, xb0, xb1, ob0, ob1, si0, si1, so0, so1):
    w = lax.axis_index("s") * 2 + lax.axis_index("c")
    iota = lax.iota(jnp.int32, 16)

    def fire_in(c, xb, sem):
        pltpu.make_async_copy(
            tabT_hbm.at[:, pl.ds(c * DT_VB, DT_VB)],
            xb.at[:, pl.ds(0, DT_VB)], sem).start()

    def wait_in(xb, sem):
        pltpu.make_async_copy(
            tabT_hbm.at[:, pl.ds(0, DT_VB)],
            xb.at[:, pl.ds(0, DT_VB)], sem).wait()

    def fire_out(c, ob, sem):
        pltpu.make_async_copy(ob, out_hbm.at[pl.ds(pl.multiple_of(c * (DT_FLAT // 2), 8), DT_FLAT // 2)], sem).start()

    def wait_out(ob, sem):
        pltpu.make_async_copy(ob, out_hbm.at[pl.ds(0, DT_FLAT // 2)], sem).wait()

    perm_idx = {s: iota ^ s for s in (1, 2, 4, 8)}
    mask_eq = {s: (iota & s) == 0 for s in (1, 2, 4, 8)}
    mask_ne = {s: (iota & s) != 0 for s in (1, 2, 4, 8)}

    def transpose16(xb, i0, e0):
        r = [xb[e0 + e, pl.ds(i0, 16)] for e in range(16)]
        for s in (1, 2, 4, 8):
            idx = perm_idx[s]
            nxt = []
            for v in range(16):
                tmp = r[v ^ s].at[idx].get(mode="promise_in_bounds")
                keep = mask_eq[s] if (v & s) == 0 else mask_ne[s]
                nxt.append(jnp.where(keep, r[v], tmp))
            r = nxt
        return r

    def transpose(xb, ob, n_i):
        def tbody(ii, carry):
            i0 = ii * 16
            lo = transpose16(xb, i0, 0)
            hi = transpose16(xb, i0, 16)
            for v in range(16):
                packed = plsc.pack(lo[v], hi[v], format=plsc.PackFormat.INTERLEAVED)
                ob[pl.ds(pl.multiple_of((i0 + v) * 16, 8), 16)] = plsc.bitcast(packed, jnp.float32)
            return carry
        lax.fori_loop(0, n_i // 16, tbody, 0)

    fire_in(w, xb0, si0)
    fire_in(w + 32, xb1, si1)

    def body(k2, carry):
        for p, xb, ob, si, so in ((0, xb0, ob0, si0, so0), (1, xb1, ob1, si1, so1)):
            k = 2 * k2 + p
            c = w + 32 * k

            @pl.when(c < DT_NFULL)
            def _():
                wait_in(xb, si)

                @pl.when(k2 > 0)
                def _():
                    wait_out(ob, so)

                transpose(xb, ob, DT_VB)
                fire_out(c, ob, so)

                @pl.when(w + 32 * (k + 2) < DT_NFULL)
                def _():
                    fire_in(w + 32 * (k + 2), xb, si)
        return carry

    lax.fori_loop(0, (DT_NFULL + 63) // 64 + 1, body, 0)
    wait_out(ob0, so0)
    wait_out(ob1, so1)

    @pl.when(w == 31)
    def _():
        cp = pltpu.make_async_copy(tail_hbm, ob0.at[pl.ds(0, DT_TAIL_V * E // 2)], si0)
        cp.start()
        cp.wait()
        cpo = pltpu.make_async_copy(
            ob0.at[pl.ds(0, DT_TAIL_V * E // 2)],
            out_hbm.at[pl.ds(DT_NFULL * DT_FLAT // 2, DT_TAIL_V * E // 2)], so0)
        cpo.start()
        cpo.wait()


R = 8
NCHUNK = BPW // R
_SPLITS = ((0, 128), (128, 72))


@functools.partial(
    pl.kernel,
    out_type=jax.ShapeDtypeStruct((B, E), jnp.float32),
    mesh=_mesh,
    scratch_types=[
        pltpu.VMEM((R, L), jnp.int32),
        pltpu.VMEM((R, L), jnp.int32),
        pltpu.VMEM((R, L, E // 2), jnp.float32),
        pltpu.VMEM((R, L, E // 2), jnp.float32),
        pltpu.VMEM((BPW, E), jnp.float32),
        pltpu.SemaphoreType.DMA,
        pltpu.SemaphoreType.DMA,
        pltpu.SemaphoreType.DMA,
    ],
    compiler_params=pltpu.CompilerParams(
        use_tc_tiling_on_sc=False, needs_layout_passes=False),
)
def _pool_sc(ids_hbm, table_hbm, out_hbm, ibuf0, ibuf1, rbuf0, rbuf1,
             obuf, sem_i, sem_g0, sem_g1):
    wid = lax.axis_index("s") * 2 + lax.axis_index("c")
    base = wid * BPW

    def fire_idx(c, ibuf):
        pltpu.make_async_copy(ids_hbm.at[pl.ds(base + c * R, R)], ibuf, sem_i).start()

    def wait_idx(ibuf):
        pltpu.make_async_copy(ids_hbm.at[pl.ds(base, R)], ibuf, sem_i).wait()

    def fire_gathers(ibuf, rbuf, sem):
        for r in range(R):
            for (o, w) in _SPLITS:
                pltpu.make_async_copy(
                    table_hbm.at[ibuf.at[r, pl.ds(o, w)]],
                    rbuf.at[r, pl.ds(o, w)], sem).start()

    def wait_gathers(ibuf, rbuf, sem):
        for r in range(R):
            for (o, w) in _SPLITS:
                pltpu.make_async_copy(
                    table_hbm.at[ibuf.at[r, pl.ds(o, w)]],
                    rbuf.at[r, pl.ds(o, w)], sem).wait()

    def reduce_chunk(c, rbuf):
        def red(j, accs):
            out = []
            for r in range(R):
                a, b = plsc.unpack(plsc.bitcast(rbuf[r, j, :], jnp.bfloat16), format=plsc.PackFormat.INTERLEAVED)
                out.append(accs[2 * r] + a)
                out.append(accs[2 * r + 1] + b)
            return tuple(out)
        z = jnp.zeros((16,), jnp.float32)
        accs = lax.fori_loop(0, L, red, (z,) * (2 * R))
        for r in range(R):
            row = c * R + r
            obuf[row, pl.ds(0, 16)] = accs[2 * r]
            obuf[row, pl.ds(16, 16)] = accs[2 * r + 1]

    fire_idx(0, ibuf0)

    def body(c2, carry):
        c = 2 * c2
        wait_idx(ibuf0)
        fire_gathers(ibuf0, rbuf0, sem_g0)

        @pl.when(c2 > 0)
        def _():
            wait_gathers(ibuf1, rbuf1, sem_g1)

        fire_idx(c + 1, ibuf1)

        @pl.when(c2 > 0)
        def _():
            reduce_chunk(c - 1, rbuf1)

        wait_idx(ibuf1)
        fire_gathers(ibuf1, rbuf1, sem_g1)
        wait_gathers(ibuf0, rbuf0, sem_g0)

        @pl.when(c2 < NCHUNK // 2 - 1)
        def _():
            fire_idx(c + 2, ibuf0)

        reduce_chunk(c, rbuf0)
        return carry

    lax.fori_loop(0, NCHUNK // 2, body, 0)
    wait_gathers(ibuf1, rbuf1, sem_g1)
    reduce_chunk(NCHUNK - 1, rbuf1)
    pltpu.sync_copy(obuf, out_hbm.at[pl.ds(base, BPW)])


def _mlp_body(x_ref, w1_ref, b1_ref, w2_ref, b2_ref, o_ref):
    h = jnp.dot(x_ref[...], w1_ref[...], preferred_element_type=jnp.float32)
    h = jnp.maximum(h + b1_ref[...], 0.0)
    o = jnp.dot(h, w2_ref[...], preferred_element_type=jnp.float32) + b2_ref[...]
    o_ref[...] = o[:, :NCLS]


_BM = 2048

_mlp = pl.pallas_call(
    _mlp_body,
    grid=(B // _BM,),
    in_specs=[
        pl.BlockSpec((_BM, E), lambda i: (i, 0)),
        pl.BlockSpec((E, 128), lambda i: (0, 0)),
        pl.BlockSpec((1, 128), lambda i: (0, 0)),
        pl.BlockSpec((128, NCLS_PAD), lambda i: (0, 0)),
        pl.BlockSpec((1, NCLS_PAD), lambda i: (0, 0)),
    ],
    out_specs=pl.BlockSpec((_BM, NCLS), lambda i: (i, 0)),
    out_shape=jax.ShapeDtypeStruct((B, NCLS), jnp.float32),
)


def kernel(input_ids, table, W1, b1, W2, b2):
    perm = jnp.stack([jnp.arange(16), jnp.arange(16) + 16], axis=1).reshape(-1)
    tail_bf = table[DT_NFULL * DT_VB:][:, perm].astype(jnp.bfloat16)
    tail = jax.lax.bitcast_convert_type(tail_bf.reshape(-1, 2), jnp.float32).reshape(-1)
    lin = _detile_sc(table.T, tail)
    pooled = _pool_sc(input_ids.astype(jnp.int32), lin.reshape(V, E // 2))
    w1s = W1.T.astype(jnp.float32) * (1.0 / L)
    b1r = b1.reshape(1, 128)
    w2p = jnp.pad(W2.T, ((0, 0), (0, NCLS_PAD - NCLS)))
    b2p = jnp.pad(b2, (0, NCLS_PAD - NCLS)).reshape(1, NCLS_PAD)
    return _mlp(pooled, w1s, b1r, w2p, b2p)


# speedup vs baseline: 3.5568x; 1.0007x over previous
import functools

import jax
import jax.numpy as jnp
from jax import lax
from jax.experimental import pallas as pl
from jax.experimental.pallas import tpu as pltpu
from jax.experimental.pallas import tpu_sc as plsc

B = 16384
L = 200
E = 32
V = 1000000
NCLS = 50
NCLS_PAD = 64
NW = 32
BPW = B // NW

_mesh = plsc.VectorSubcoreMesh(core_axis_name="c", subcore_axis_name="s")

DT_CR = 192
DT_VB = 4 * DT_CR
DT_FLAT = DT_VB * E
DT_NFULL = (V // 4) // DT_CR
DT_TAIL_V = V - DT_NFULL * DT_VB
_XBP = 775


@functools.partial(
    pl.kernel,
    out_type=jax.ShapeDtypeStruct((V * E // 2,), jnp.float32),
    mesh=_mesh,
    scratch_types=[
        pltpu.VMEM((32, _XBP), jnp.float32),
        pltpu.VMEM((32, _XBP), jnp.float32),
        pltpu.VMEM((DT_FLAT // 2,), jnp.float32),
        pltpu.VMEM((DT_FLAT // 2,), jnp.float32),
        pltpu.SemaphoreType.DMA,
        pltpu.SemaphoreType.DMA,
        pltpu.SemaphoreType.DMA,
        pltpu.SemaphoreType.DMA,
    ],
    compiler_params=pltpu.CompilerParams(
        use_tc_tiling_on_sc=True, needs_layout_passes=False),
)
def _detile_sc(tabT_hbm, tail_hbm, out_hbm, xb0, xb1, ob0, ob1, si0, si1, so0, so1):
    w = lax.axis_index("s") * 2 + lax.axis_index("c")
    iota = lax.iota(jnp.int32, 16)

    def fire_in(c, xb, sem):
        pltpu.make_async_copy(
            tabT_hbm.at[:, pl.ds(c * DT_VB, DT_VB)],
            xb.at[:, pl.ds(0, DT_VB)], sem).start()

    def wait_in(xb, sem):
        pltpu.make_async_copy(
            tabT_hbm.at[:, pl.ds(0, DT_VB)],
            xb.at[:, pl.ds(0, DT_VB)], sem).wait()

    def fire_out(c, ob, sem):
        pltpu.make_async_copy(ob, out_hbm.at[pl.ds(pl.multiple_of(c * (DT_FLAT // 2), 8), DT_FLAT // 2)], sem).start()

    def wait_out(ob, sem):
        pltpu.make_async_copy(ob, out_hbm.at[pl.ds(0, DT_FLAT // 2)], sem).wait()

    perm_idx = {s: iota ^ s for s in (1, 2, 4, 8)}
    mask_eq = {s: (iota & s) == 0 for s in (1, 2, 4, 8)}
    mask_ne = {s: (iota & s) != 0 for s in (1, 2, 4, 8)}

    def transpose16(xb, i0, e0):
        r = [xb[e0 + e, pl.ds(i0, 16)] for e in range(16)]
        for s in (1, 2, 4, 8):
            idx = perm_idx[s]
            nxt = []
            for v in range(16):
                tmp = r[v ^ s].at[idx].get(mode="promise_in_bounds")
                keep = mask_eq[s] if (v & s) == 0 else mask_ne[s]
                nxt.append(jnp.where(keep, r[v], tmp))
            r = nxt
        return r

    def transpose(xb, ob, n_i):
        def tbody(ii, carry):
            i0 = ii * 16
            lo = transpose16(xb, i0, 0)
            hi = transpose16(xb, i0, 16)
            for v in range(16):
                packed = plsc.pack(lo[v], hi[v], format=plsc.PackFormat.INTERLEAVED)
                ob[pl.ds(pl.multiple_of((i0 + v) * 16, 8), 16)] = plsc.bitcast(packed, jnp.float32)
            return carry
        lax.fori_loop(0, n_i // 16, tbody, 0)

    fire_in(w, xb0, si0)
    fire_in(w + 32, xb1, si1)

    def body(k2, carry):
        for p, xb, ob, si, so in ((0, xb0, ob0, si0, so0), (1, xb1, ob1, si1, so1)):
            k = 2 * k2 + p
            c = w + 32 * k

            @pl.when(c < DT_NFULL)
            def _():
                wait_in(xb, si)

                @pl.when(k2 > 0)
                def _():
                    wait_out(ob, so)

                transpose(xb, ob, DT_VB)
                fire_out(c, ob, so)

                @pl.when(w + 32 * (k + 2) < DT_NFULL)
                def _():
                    fire_in(w + 32 * (k + 2), xb, si)
        return carry

    lax.fori_loop(0, (DT_NFULL + 63) // 64 + 1, body, 0)
    wait_out(ob0, so0)
    wait_out(ob1, so1)

    @pl.when(w == 31)
    def _():
        cp = pltpu.make_async_copy(tail_hbm, ob0.at[pl.ds(0, DT_TAIL_V * E // 2)], si0)
        cp.start()
        cp.wait()
        cpo = pltpu.make_async_copy(
            ob0.at[pl.ds(0, DT_TAIL_V * E // 2)],
            out_hbm.at[pl.ds(DT_NFULL * DT_FLAT // 2, DT_TAIL_V * E // 2)], so0)
        cpo.start()
        cpo.wait()


R = 8
NCHUNK = BPW // R
_SPLITS = ((0, 128), (128, 72))


@functools.partial(
    pl.kernel,
    out_type=jax.ShapeDtypeStruct((B, E), jnp.float32),
    mesh=_mesh,
    scratch_types=[
        pltpu.VMEM((R, L), jnp.int32),
        pltpu.VMEM((R, L), jnp.int32),
        pltpu.VMEM((R, L, E // 2), jnp.float32),
        pltpu.VMEM((R, L, E // 2), jnp.float32),
        pltpu.VMEM((BPW, E), jnp.float32),
        pltpu.SemaphoreType.DMA,
        pltpu.SemaphoreType.DMA,
        pltpu.SemaphoreType.DMA,
    ],
    compiler_params=pltpu.CompilerParams(
        use_tc_tiling_on_sc=False, needs_layout_passes=False),
)
def _pool_sc(ids_hbm, table_hbm, out_hbm, ibuf0, ibuf1, rbuf0, rbuf1,
             obuf, sem_i, sem_g0, sem_g1):
    wid = lax.axis_index("s") * 2 + lax.axis_index("c")
    base = wid * BPW

    def fire_idx(c, ibuf):
        pltpu.make_async_copy(ids_hbm.at[pl.ds(base + c * R, R)], ibuf, sem_i).start()

    def wait_idx(ibuf):
        pltpu.make_async_copy(ids_hbm.at[pl.ds(base, R)], ibuf, sem_i).wait()

    def fire_gathers(ibuf, rbuf, sem):
        for r in range(R):
            for (o, w) in _SPLITS:
                pltpu.make_async_copy(
                    table_hbm.at[ibuf.at[r, pl.ds(o, w)]],
                    rbuf.at[r, pl.ds(o, w)], sem).start()

    def wait_gathers(ibuf, rbuf, sem):
        for r in range(R):
            for (o, w) in _SPLITS:
                pltpu.make_async_copy(
                    table_hbm.at[ibuf.at[r, pl.ds(o, w)]],
                    rbuf.at[r, pl.ds(o, w)], sem).wait()

    def reduce_chunk(c, rbuf):
        himask = jnp.full((16,), -65536, jnp.int32)

        def red(j, accs):
            out = []
            for r in range(R):
                xi = plsc.bitcast(rbuf[r, j, :], jnp.int32)
                a = plsc.bitcast(xi << 16, jnp.float32)
                b = plsc.bitcast(xi & himask, jnp.float32)
                out.append(accs[2 * r] + a)
                out.append(accs[2 * r + 1] + b)
            return tuple(out)
        z = jnp.zeros((16,), jnp.float32)
        accs = lax.fori_loop(0, L, red, (z,) * (2 * R))
        for r in range(R):
            row = c * R + r
            obuf[row, pl.ds(0, 16)] = accs[2 * r]
            obuf[row, pl.ds(16, 16)] = accs[2 * r + 1]

    fire_idx(0, ibuf0)

    def body(c2, carry):
        c = 2 * c2
        wait_idx(ibuf0)
        fire_gathers(ibuf0, rbuf0, sem_g0)

        @pl.when(c2 > 0)
        def _():
            wait_gathers(ibuf1, rbuf1, sem_g1)

        fire_idx(c + 1, ibuf1)

        @pl.when(c2 > 0)
        def _():
            reduce_chunk(c - 1, rbuf1)

        wait_idx(ibuf1)
        fire_gathers(ibuf1, rbuf1, sem_g1)
        wait_gathers(ibuf0, rbuf0, sem_g0)

        @pl.when(c2 < NCHUNK // 2 - 1)
        def _():
            fire_idx(c + 2, ibuf0)

        reduce_chunk(c, rbuf0)
        return carry

    lax.fori_loop(0, NCHUNK // 2, body, 0)
    wait_gathers(ibuf1, rbuf1, sem_g1)
    reduce_chunk(NCHUNK - 1, rbuf1)
    pltpu.sync_copy(obuf, out_hbm.at[pl.ds(base, BPW)])


def _mlp_body(x_ref, w1_ref, b1_ref, w2_ref, b2_ref, o_ref):
    h = jnp.dot(x_ref[...], w1_ref[...], preferred_element_type=jnp.float32)
    h = jnp.maximum(h + b1_ref[...], 0.0)
    o = jnp.dot(h, w2_ref[...], preferred_element_type=jnp.float32) + b2_ref[...]
    o_ref[...] = o[:, :NCLS]


_BM = 2048

_mlp = pl.pallas_call(
    _mlp_body,
    grid=(B // _BM,),
    in_specs=[
        pl.BlockSpec((_BM, E), lambda i: (i, 0)),
        pl.BlockSpec((E, 128), lambda i: (0, 0)),
        pl.BlockSpec((1, 128), lambda i: (0, 0)),
        pl.BlockSpec((128, NCLS_PAD), lambda i: (0, 0)),
        pl.BlockSpec((1, NCLS_PAD), lambda i: (0, 0)),
    ],
    out_specs=pl.BlockSpec((_BM, NCLS), lambda i: (i, 0)),
    out_shape=jax.ShapeDtypeStruct((B, NCLS), jnp.float32),
)


def kernel(input_ids, table, W1, b1, W2, b2):
    perm = jnp.stack([jnp.arange(16), jnp.arange(16) + 16], axis=1).reshape(-1)
    tail_bf = table[DT_NFULL * DT_VB:][:, perm].astype(jnp.bfloat16)
    tail = jax.lax.bitcast_convert_type(tail_bf.reshape(-1, 2), jnp.float32).reshape(-1)
    lin = _detile_sc(table.T, tail)
    pooled = _pool_sc(input_ids.astype(jnp.int32), lin.reshape(V, E // 2))
    w1s = W1.T.astype(jnp.float32) * (1.0 / L)
    b1r = b1.reshape(1, 128)
    w2p = jnp.pad(W2.T, ((0, 0), (0, NCLS_PAD - NCLS)))
    b2p = jnp.pad(b2, (0, NCLS_PAD - NCLS)).reshape(1, NCLS_PAD)
    return _mlp(pooled, w1s, b1r, w2p, b2p)


# speedup vs baseline: 3.5572x; 1.0001x over previous
import functools

import jax
import jax.numpy as jnp
from jax import lax
from jax.experimental import pallas as pl
from jax.experimental.pallas import tpu as pltpu
from jax.experimental.pallas import tpu_sc as plsc

B = 16384
L = 200
E = 32
V = 1000000
NCLS = 50
NCLS_PAD = 64
NW = 32
BPW = B // NW

_mesh = plsc.VectorSubcoreMesh(core_axis_name="c", subcore_axis_name="s")

DT_CR = 192
DT_VB = 4 * DT_CR
DT_FLAT = DT_VB * E
DT_NFULL = (V // 4) // DT_CR
DT_TAIL_V = V - DT_NFULL * DT_VB
_XBP = 775


@functools.partial(
    pl.kernel,
    out_type=jax.ShapeDtypeStruct((V * E // 2,), jnp.float32),
    mesh=_mesh,
    scratch_types=[
        pltpu.VMEM((32, _XBP), jnp.float32),
        pltpu.VMEM((32, _XBP), jnp.float32),
        pltpu.VMEM((DT_FLAT // 2,), jnp.float32),
        pltpu.VMEM((DT_FLAT // 2,), jnp.float32),
        pltpu.SemaphoreType.DMA,
        pltpu.SemaphoreType.DMA,
        pltpu.SemaphoreType.DMA,
        pltpu.SemaphoreType.DMA,
    ],
    compiler_params=pltpu.CompilerParams(
        use_tc_tiling_on_sc=True, needs_layout_passes=False),
)
def _detile_sc(tabT_hbm, tail_hbm, out_hbm, xb0, xb1, ob0, ob1, si0, si1, so0, so1):
    w = lax.axis_index("s") * 2 + lax.axis_index("c")
    iota = lax.iota(jnp.int32, 16)

    def fire_in(c, xb, sem):
        pltpu.make_async_copy(
            tabT_hbm.at[:, pl.ds(c * DT_VB, DT_VB)],
            xb.at[:, pl.ds(0, DT_VB)], sem).start()

    def wait_in(xb, sem):
        pltpu.make_async_copy(
            tabT_hbm.at[:, pl.ds(0, DT_VB)],
            xb.at[:, pl.ds(0, DT_VB)], sem).wait()

    def fire_out(c, ob, sem):
        pltpu.make_async_copy(ob, out_hbm.at[pl.ds(pl.multiple_of(c * (DT_FLAT // 2), 8), DT_FLAT // 2)], sem).start()

    def wait_out(ob, sem):
        pltpu.make_async_copy(ob, out_hbm.at[pl.ds(0, DT_FLAT // 2)], sem).wait()

    perm_idx = {s: iota ^ s for s in (1, 2, 4, 8)}
    mask_eq = {s: (iota & s) == 0 for s in (1, 2, 4, 8)}
    mask_ne = {s: (iota & s) != 0 for s in (1, 2, 4, 8)}

    def transpose16(xb, i0, e0):
        r = [xb[e0 + e, pl.ds(i0, 16)] for e in range(16)]
        for s in (1, 2, 4, 8):
            idx = perm_idx[s]
            nxt = []
            for v in range(16):
                tmp = r[v ^ s].at[idx].get(mode="promise_in_bounds")
                keep = mask_eq[s] if (v & s) == 0 else mask_ne[s]
                nxt.append(jnp.where(keep, r[v], tmp))
            r = nxt
        return r

    def transpose(xb, ob, n_i):
        def tbody(ii, carry):
            i0 = ii * 16
            lo = transpose16(xb, i0, 0)
            hi = transpose16(xb, i0, 16)
            for v in range(16):
                packed = plsc.pack(lo[v], hi[v], format=plsc.PackFormat.INTERLEAVED)
                ob[pl.ds(pl.multiple_of((i0 + v) * 16, 8), 16)] = plsc.bitcast(packed, jnp.float32)
            return carry
        lax.fori_loop(0, n_i // 16, tbody, 0)

    fire_in(w, xb0, si0)
    fire_in(w + 32, xb1, si1)

    def body(k2, carry):
        for p, xb, ob, si, so in ((0, xb0, ob0, si0, so0), (1, xb1, ob1, si1, so1)):
            k = 2 * k2 + p
            c = w + 32 * k

            @pl.when(c < DT_NFULL)
            def _():
                wait_in(xb, si)

                @pl.when(k2 > 0)
                def _():
                    wait_out(ob, so)

                transpose(xb, ob, DT_VB)
                fire_out(c, ob, so)

                @pl.when(w + 32 * (k + 2) < DT_NFULL)
                def _():
                    fire_in(w + 32 * (k + 2), xb, si)
        return carry

    lax.fori_loop(0, (DT_NFULL + 63) // 64 + 1, body, 0)
    wait_out(ob0, so0)
    wait_out(ob1, so1)

    @pl.when(w == 31)
    def _():
        cp = pltpu.make_async_copy(tail_hbm, ob0.at[pl.ds(0, DT_TAIL_V * E // 2)], si0)
        cp.start()
        cp.wait()
        cpo = pltpu.make_async_copy(
            ob0.at[pl.ds(0, DT_TAIL_V * E // 2)],
            out_hbm.at[pl.ds(DT_NFULL * DT_FLAT // 2, DT_TAIL_V * E // 2)], so0)
        cpo.start()
        cpo.wait()


R = 8
NCHUNK = BPW // R
_SPLITS = ((0, 128), (128, 72))


@functools.partial(
    pl.kernel,
    out_type=jax.ShapeDtypeStruct((B, E), jnp.float32),
    mesh=_mesh,
    scratch_types=[
        pltpu.VMEM((R * L,), jnp.int32),
        pltpu.VMEM((R * L,), jnp.int32),
        pltpu.VMEM((R, L, E // 2), jnp.float32),
        pltpu.VMEM((R, L, E // 2), jnp.float32),
        pltpu.VMEM((BPW, E), jnp.float32),
        pltpu.SemaphoreType.DMA,
        pltpu.SemaphoreType.DMA,
        pltpu.SemaphoreType.DMA,
    ],
    compiler_params=pltpu.CompilerParams(
        use_tc_tiling_on_sc=False, needs_layout_passes=False),
)
def _pool_sc(ids_hbm, table_hbm, out_hbm, ibuf0, ibuf1, rbuf0, rbuf1,
             obuf, sem_i, sem_g0, sem_g1):
    wid = lax.axis_index("s") * 2 + lax.axis_index("c")
    base = wid * BPW

    def fire_idx(c, ibuf):
        pltpu.make_async_copy(
            ids_hbm.at[pl.ds(pl.multiple_of((base + c * R) * L, 8), R * L)],
            ibuf, sem_i).start()

    def wait_idx(ibuf):
        pltpu.make_async_copy(ids_hbm.at[pl.ds(0, R * L)], ibuf, sem_i).wait()

    def fire_gathers(ibuf, rbuf, sem):
        for r in range(R):
            for (o, w) in _SPLITS:
                pltpu.make_async_copy(
                    table_hbm.at[ibuf.at[pl.ds(r * L + o, w)]],
                    rbuf.at[r, pl.ds(o, w)], sem).start()

    def wait_gathers(ibuf, rbuf, sem):
        for r in range(R):
            for (o, w) in _SPLITS:
                pltpu.make_async_copy(
                    table_hbm.at[ibuf.at[pl.ds(r * L + o, w)]],
                    rbuf.at[r, pl.ds(o, w)], sem).wait()

    def reduce_chunk(c, rbuf):
        himask = jnp.full((16,), -65536, jnp.int32)

        def red(j, accs):
            out = []
            for r in range(R):
                xi = plsc.bitcast(rbuf[r, j, :], jnp.int32)
                a = plsc.bitcast(xi << 16, jnp.float32)
                b = plsc.bitcast(xi & himask, jnp.float32)
                out.append(accs[2 * r] + a)
                out.append(accs[2 * r + 1] + b)
            return tuple(out)
        z = jnp.zeros((16,), jnp.float32)
        accs = lax.fori_loop(0, L, red, (z,) * (2 * R))
        for r in range(R):
            row = c * R + r
            obuf[row, pl.ds(0, 16)] = accs[2 * r]
            obuf[row, pl.ds(16, 16)] = accs[2 * r + 1]

    fire_idx(0, ibuf0)

    def body(c2, carry):
        c = 2 * c2
        wait_idx(ibuf0)
        fire_gathers(ibuf0, rbuf0, sem_g0)

        @pl.when(c2 > 0)
        def _():
            wait_gathers(ibuf1, rbuf1, sem_g1)

        fire_idx(c + 1, ibuf1)

        @pl.when(c2 > 0)
        def _():
            reduce_chunk(c - 1, rbuf1)

        wait_idx(ibuf1)
        fire_gathers(ibuf1, rbuf1, sem_g1)
        wait_gathers(ibuf0, rbuf0, sem_g0)

        @pl.when(c2 < NCHUNK // 2 - 1)
        def _():
            fire_idx(c + 2, ibuf0)

        reduce_chunk(c, rbuf0)
        return carry

    lax.fori_loop(0, NCHUNK // 2, body, 0)
    wait_gathers(ibuf1, rbuf1, sem_g1)
    reduce_chunk(NCHUNK - 1, rbuf1)
    pltpu.sync_copy(obuf, out_hbm.at[pl.ds(base, BPW)])


def _mlp_body(x_ref, w1_ref, b1_ref, w2_ref, b2_ref, o_ref):
    h = jnp.dot(x_ref[...], w1_ref[...], preferred_element_type=jnp.float32)
    h = jnp.maximum(h + b1_ref[...], 0.0)
    o = jnp.dot(h, w2_ref[...], preferred_element_type=jnp.float32) + b2_ref[...]
    o_ref[...] = o[:, :NCLS]


_BM = 2048

_mlp = pl.pallas_call(
    _mlp_body,
    grid=(B // _BM,),
    in_specs=[
        pl.BlockSpec((_BM, E), lambda i: (i, 0)),
        pl.BlockSpec((E, 128), lambda i: (0, 0)),
        pl.BlockSpec((1, 128), lambda i: (0, 0)),
        pl.BlockSpec((128, NCLS_PAD), lambda i: (0, 0)),
        pl.BlockSpec((1, NCLS_PAD), lambda i: (0, 0)),
    ],
    out_specs=pl.BlockSpec((_BM, NCLS), lambda i: (i, 0)),
    out_shape=jax.ShapeDtypeStruct((B, NCLS), jnp.float32),
)


def kernel(input_ids, table, W1, b1, W2, b2):
    perm = jnp.stack([jnp.arange(16), jnp.arange(16) + 16], axis=1).reshape(-1)
    tail_bf = table[DT_NFULL * DT_VB:][:, perm].astype(jnp.bfloat16)
    tail = jax.lax.bitcast_convert_type(tail_bf.reshape(-1, 2), jnp.float32).reshape(-1)
    lin = _detile_sc(table.T, tail)
    pooled = _pool_sc(input_ids.astype(jnp.int32).reshape(-1), lin.reshape(V, E // 2))
    w1s = W1.T.astype(jnp.float32) * (1.0 / L)
    b1r = b1.reshape(1, 128)
    w2p = jnp.pad(W2.T, ((0, 0), (0, NCLS_PAD - NCLS)))
    b2p = jnp.pad(b2, (0, NCLS_PAD - NCLS)).reshape(1, NCLS_PAD)
    return _mlp(pooled, w1s, b1r, w2p, b2p)
